# baseline (speedup 1.0000x reference)
import functools

import jax
import jax.numpy as jnp
from jax import lax
from jax.experimental import pallas as pl
from jax.experimental.pallas import tpu as pltpu
from jax.experimental.pallas import tpu_sc as plsc

E = 64
T = 8192
D = 1024
DFF = 2048
BM = 256
NTP = T // BM + E
TP = NTP * BM


def _sc_place_rows(table, src_idx, dst_idx, n_out):
    info = plsc.get_sparse_core_info()
    nw = info.num_cores * info.num_subcores
    b = src_idx.shape[0]
    d = table.shape[1]
    b_per_w = b // nw
    ch = 64
    n_ch = b_per_w // ch
    mesh = plsc.VectorSubcoreMesh(core_axis_name="c", subcore_axis_name="s")

    @functools.partial(
        pl.kernel,
        mesh=mesh,
        out_type=jax.ShapeDtypeStruct((n_out, d), jnp.float32),
        scratch_types=[
            pltpu.VMEM((ch,), jnp.int32),
            pltpu.VMEM((ch,), jnp.int32),
            pltpu.VMEM((ch, d), jnp.float32),
            pltpu.SemaphoreType.DMA,
            pltpu.SemaphoreType.DMA,
        ],
    )
    def k(table_hbm, src_hbm, dst_hbm, out_hbm, src_v, dst_v, rows_v,
          sem_g, sem_s):
        wid = lax.axis_index("s") * info.num_cores + lax.axis_index("c")
        base = wid * b_per_w

        def body(i, carry):
            off = base + i * ch
            pltpu.sync_copy(src_hbm.at[pl.ds(off, ch)], src_v)
            pltpu.sync_copy(dst_hbm.at[pl.ds(off, ch)], dst_v)
            pltpu.async_copy(table_hbm.at[src_v], rows_v, sem_g).wait()
            pltpu.async_copy(rows_v, out_hbm.at[dst_v], sem_s).wait()
            return carry

        lax.fori_loop(0, n_ch, body, 0)

    return k(table, src_idx, dst_idx)


def _sc_gather_rows(table, idx, n_out):
    info = plsc.get_sparse_core_info()
    nw = info.num_cores * info.num_subcores
    b = idx.shape[0]
    d = table.shape[1]
    b_per_w = b // nw
    ch = 64
    n_ch = b_per_w // ch
    mesh = plsc.VectorSubcoreMesh(core_axis_name="c", subcore_axis_name="s")

    @functools.partial(
        pl.kernel,
        mesh=mesh,
        out_type=jax.ShapeDtypeStruct((n_out, d), jnp.float32),
        scratch_types=[
            pltpu.VMEM((ch,), jnp.int32),
            pltpu.VMEM((ch, d), jnp.float32),
            pltpu.SemaphoreType.DMA,
        ],
    )
    def k(table_hbm, idx_hbm, out_hbm, idx_v, rows_v, sem):
        wid = lax.axis_index("s") * info.num_cores + lax.axis_index("c")
        base = wid * b_per_w

        def body(i, carry):
            off = base + i * ch
            pltpu.sync_copy(idx_hbm.at[pl.ds(off, ch)], idx_v)
            pltpu.async_copy(table_hbm.at[idx_v], rows_v, sem).wait()
            pltpu.sync_copy(rows_v, out_hbm.at[pl.ds(off, ch)])
            return carry

        lax.fori_loop(0, n_ch, body, 0)

    return k(table, idx)


def _ffn_body(e_ref, chg_ref, slot_ref, nxt_ref, has_ref, val_ref, m_ref,
              x_ref, gu_hbm, dp_hbm, c_ref, o_ref,
              gu_buf, dp_buf, gu_sem, dp_sem):
    t = pl.program_id(0)
    chg = chg_ref[t]
    slot = slot_ref[t]

    def start_fetch(e_idx, s):
        pltpu.make_async_copy(gu_hbm.at[e_idx], gu_buf.at[s],
                              gu_sem.at[s]).start()
        pltpu.make_async_copy(dp_hbm.at[e_idx], dp_buf.at[s],
                              dp_sem.at[s]).start()

    @pl.when(t == 0)
    def _():
        start_fetch(e_ref[0], 0)

    @pl.when(jnp.logical_and(chg == 1, has_ref[t] == 1))
    def _():
        start_fetch(nxt_ref[t], 1 - slot)

    @pl.when(chg == 1)
    def _():
        pltpu.make_async_copy(gu_hbm.at[0], gu_buf.at[slot],
                              gu_sem.at[slot]).wait()
        pltpu.make_async_copy(dp_hbm.at[0], dp_buf.at[slot],
                              dp_sem.at[slot]).wait()

    @pl.when(val_ref[t] == 1)
    def _():
        x = x_ref[...].astype(jnp.bfloat16)
        c = c_ref[0, 0, :][:, None]
        dn = (((1,), (1,)), ((), ()))
        gu = lax.dot_general(x, gu_buf[slot].astype(jnp.bfloat16), dn,
                             preferred_element_type=jnp.float32)
        gate = gu[:, :DFF]
        up = gu[:, DFF:]
        act = (gate * lax.logistic(gate) * up * c).astype(jnp.bfloat16)
        o_ref[...] = lax.dot_general(act, dp_buf[slot].astype(jnp.bfloat16),
                                     dn, preferred_element_type=jnp.float32)


def _tc_tiled_ffn(x_pad, gate_up_proj, down_proj,
                  e_sched, chg, slot, nxt, has, valid, m_sched, coeffs):
    grid_spec = pltpu.PrefetchScalarGridSpec(
        num_scalar_prefetch=7,
        grid=(NTP,),
        in_specs=[
            pl.BlockSpec((BM, D), lambda t, e, c_, s, n, h, v, m: (m[t], 0)),
            pl.BlockSpec(memory_space=pl.ANY),
            pl.BlockSpec(memory_space=pl.ANY),
            pl.BlockSpec((1, 1, BM), lambda t, *_: (t, 0, 0)),
        ],
        out_specs=pl.BlockSpec(
            (BM, D), lambda t, e, c_, s, n, h, v, m: (m[t], 0)),
        scratch_shapes=[
            pltpu.VMEM((2, 2 * DFF, D), jnp.float32),
            pltpu.VMEM((2, D, DFF), jnp.float32),
            pltpu.SemaphoreType.DMA((2,)),
            pltpu.SemaphoreType.DMA((2,)),
        ],
    )
    return pl.pallas_call(
        _ffn_body,
        grid_spec=grid_spec,
        out_shape=jax.ShapeDtypeStruct((TP, D), jnp.float32),
        compiler_params=pltpu.CompilerParams(
            dimension_semantics=("arbitrary",),
        ),
    )(e_sched, chg, slot, nxt, has, valid, m_sched,
      x_pad, gate_up_proj, down_proj, coeffs)


def _build_schedule(idx, w):
    order = jnp.argsort(idx).astype(jnp.int32)
    sids = idx[order]
    wsort = w[order]

    er = jnp.arange(E, dtype=jnp.int32)
    off = jnp.searchsorted(sids, er, side="left").astype(jnp.int32)
    off_end = jnp.searchsorted(sids, er, side="right").astype(jnp.int32)
    sz = off_end - off
    tiles = (sz + BM - 1) // BM
    tstart = jnp.concatenate([jnp.zeros((1,), jnp.int32),
                              jnp.cumsum(tiles)]).astype(jnp.int32)
    total_tiles = tstart[E]

    i = jnp.arange(T, dtype=jnp.int32)
    pad_pos = tstart[sids] * BM + (i - off[sids])

    inv_pad = jnp.zeros((T,), jnp.int32).at[order].set(pad_pos)

    tt = jnp.arange(NTP, dtype=jnp.int32)
    e_tile = (jnp.searchsorted(tstart, tt, side="right") - 1).astype(jnp.int32)
    e_tile = jnp.clip(e_tile, 0, E - 1)
    valid = (tt < total_tiles).astype(jnp.int32)
    last_e = e_tile[jnp.maximum(total_tiles - 1, 0)]
    e_sched = jnp.where(valid == 1, e_tile, last_e).astype(jnp.int32)
    m_sched = jnp.where(valid == 1, tt, jnp.maximum(total_tiles - 1, 0))
    m_sched = m_sched.astype(jnp.int32)

    cf = jnp.zeros((TP,), jnp.float32).at[pad_pos].set(wsort)
    coeffs = cf.reshape(NTP, 1, BM)

    chg = jnp.concatenate([jnp.ones((1,), jnp.int32),
                           (e_sched[1:] != e_sched[:-1]).astype(jnp.int32)])
    slot = ((jnp.cumsum(chg) - 1) % 2).astype(jnp.int32)
    cand = jnp.where(chg == 1, tt, NTP)
    suf = lax.cummin(cand[::-1])[::-1]
    nxt_pos = jnp.concatenate([suf[1:], jnp.full((1,), NTP, jnp.int32)])
    has = (nxt_pos < NTP).astype(jnp.int32)
    nxt = e_sched[jnp.clip(nxt_pos, 0, NTP - 1)]

    return (order, pad_pos, inv_pad,
            e_sched, chg, slot, nxt, has, valid, m_sched, coeffs)


def kernel(hidden_states, top_k_index, top_k_weights, gate_up_proj, down_proj):
    idx = top_k_index[:, 0].astype(jnp.int32)
    w = top_k_weights[:, 0].astype(jnp.float32)

    (order, pad_pos, inv_pad, e_sched, chg, slot, nxt, has,
     valid, m_sched, coeffs) = _build_schedule(idx, w)

    x_pad = _sc_place_rows(hidden_states, order, pad_pos, TP)
    out_pad = _tc_tiled_ffn(x_pad, gate_up_proj, down_proj,
                            e_sched, chg, slot, nxt, has,
                            valid, m_sched, coeffs)
    return _sc_gather_rows(out_pad, inv_pad, T)
